# baseline (speedup 1.0000x reference)
import functools

import jax
import jax.numpy as jnp
from jax import lax
from jax.experimental import pallas as pl
from jax.experimental.pallas import tpu as pltpu
from jax.experimental.pallas import tpu_sc as plsc

NW = 32
C_OUT = 64
CHUNK = 128
SUP = 2048
SUP_LOG2 = 11
NCH = SUP // CHUNK
NEG = float("-inf")


def _mlp_body(pf_ref, xyz_ref, wa_ref, wb_ref, a_ref):
    a_ref[...] = (
        lax.dot_general(pf_ref[...], wa_ref[...], (((1,), (0,)), ((), ())),
                        precision=lax.Precision.HIGHEST,
                        preferred_element_type=jnp.float32)
        + lax.dot_general(xyz_ref[...], wb_ref[...], (((1,), (0,)), ((), ())),
                          precision=lax.Precision.HIGHEST,
                          preferred_element_type=jnp.float32))


def _run_mlp(pf4, xyz4, wa_blk, wb_blk):
    n4 = pf4.shape[0]
    bs = 2000
    return pl.pallas_call(
        _mlp_body,
        grid=(n4 // bs,),
        in_specs=[
            pl.BlockSpec((bs, 116), lambda i: (i, 0)),
            pl.BlockSpec((bs, 12), lambda i: (i, 0)),
            pl.BlockSpec((116, 256), lambda i: (0, 0)),
            pl.BlockSpec((12, 256), lambda i: (0, 0)),
        ],
        out_specs=pl.BlockSpec((bs, 256), lambda i: (i, 0)),
        out_shape=jax.ShapeDtypeStruct((n4, 256), jnp.float32),
    )(pf4, xyz4, wa_blk, wb_blk)


def _make_segmax(n_pts, l_pad, mw, m_pad):
    mesh = plsc.VectorSubcoreMesh(core_axis_name="c", subcore_axis_name="s")

    @functools.partial(
        pl.kernel,
        out_type=jax.ShapeDtypeStruct((m_pad, C_OUT), jnp.float32),
        mesh=mesh,
        compiler_params=pltpu.CompilerParams(use_tc_tiling_on_sc=False),
        scratch_types=[
            pltpu.VMEM((mw + 1, C_OUT), jnp.float32),
            pltpu.VMEM((SUP,), jnp.int32),
            pltpu.VMEM((SUP,), jnp.int32),
            pltpu.VMEM((2, CHUNK, C_OUT), jnp.float32),
            pltpu.VMEM((48,), jnp.int32),
            pltpu.SemaphoreType.DMA,
            pltpu.SemaphoreType.DMA,
            pltpu.SemaphoreType.DMA,
        ],
    )
    def segmax(a_hbm, pidx_hbm, psi_hbm, bounds_hbm, s_hbm,
               acc, pidx_v, psi_v, rows_v, bounds_v, sem_i, sem_g0, sem_g1):
        wid = lax.axis_index("s") * 2 + lax.axis_index("c")
        m0 = wid * mw
        sem_g = (sem_g0, sem_g1)

        neg16 = jnp.full((16,), NEG, jnp.float32)

        def init_row(r, carry):
            for c in range(4):
                acc[r, pl.ds(c * 16, 16)] = neg16
            return carry
        lax.fori_loop(0, mw + 1, init_row, 0, unroll=False)

        pltpu.sync_copy(bounds_hbm, bounds_v)
        bv = bounds_v[pl.ds(wid, 16)]
        lo = bv[0]
        hi = bv[1]
        lo8 = lo & jnp.int32(-8)
        nsup = jnp.maximum((hi - lo8 + (SUP - 1)) >> SUP_LOG2, 1)

        def issue_ids(s):
            base = pl.multiple_of(lo8 + s * SUP, 8)
            pltpu.async_copy(pidx_hbm.at[pl.ds(base, SUP)], pidx_v, sem_i)
            pltpu.async_copy(psi_hbm.at[pl.ds(base, SUP)], psi_v, sem_i)

        def wait_ids():
            pltpu.make_async_copy(pidx_hbm.at[pl.ds(0, SUP)], pidx_v, sem_i).wait()
            pltpu.make_async_copy(psi_hbm.at[pl.ds(0, SUP)], psi_v, sem_i).wait()

        def issue_gather(t, b):
            pltpu.async_copy(a_hbm.at[pidx_v.at[pl.ds(t * CHUNK, CHUNK)]],
                             rows_v.at[b], sem_g[b])

        def wait_gather(b):
            pltpu.make_async_copy(a_hbm.at[pidx_v.at[pl.ds(0, CHUNK)]],
                                  rows_v.at[b], sem_g[b]).wait()

        def compute_chunk(t, b, carry):
            def group(gi, c2):
                rp, m_0, m_1, m_2, m_3 = c2
                sv = psi_v[pl.ds(t * CHUNK + gi * 16, 16)] - m0
                rv = jnp.where((sv < 0) | (sv >= mw), mw, sv)
                for j in range(16):
                    r = rv[j]
                    i = gi * 16 + j
                    row0 = rows_v[b, i, pl.ds(0, 16)]
                    row1 = rows_v[b, i, pl.ds(16, 16)]
                    row2 = rows_v[b, i, pl.ds(32, 16)]
                    row3 = rows_v[b, i, pl.ds(48, 16)]
                    new = r != rp
                    m_0 = jnp.maximum(jnp.where(new, neg16, m_0), row0)
                    m_1 = jnp.maximum(jnp.where(new, neg16, m_1), row1)
                    m_2 = jnp.maximum(jnp.where(new, neg16, m_2), row2)
                    m_3 = jnp.maximum(jnp.where(new, neg16, m_3), row3)
                    acc[r, pl.ds(0, 16)] = m_0
                    acc[r, pl.ds(16, 16)] = m_1
                    acc[r, pl.ds(32, 16)] = m_2
                    acc[r, pl.ds(48, 16)] = m_3
                    rp = r
                return rp, m_0, m_1, m_2, m_3
            return lax.fori_loop(0, CHUNK // 16, group, carry, unroll=False)

        issue_ids(jnp.int32(0))

        def sup_body(s, carry):
            wait_ids()
            issue_gather(0, 0)
            for t in range(NCH):
                if t + 1 < NCH:
                    issue_gather(t + 1, (t + 1) % 2)
                wait_gather(t % 2)
                carry2 = compute_chunk(t, t % 2, carry if t == 0 else carry2)
                carry = carry2

            @pl.when(s + 1 < nsup)
            def _():
                issue_ids(s + 1)
            return carry

        carry0 = (jnp.int32(mw), neg16, neg16, neg16, neg16)
        lax.fori_loop(0, nsup, sup_body, carry0, unroll=False)

        pltpu.sync_copy(acc.at[pl.ds(0, mw)], s_hbm.at[pl.ds(m0, mw)])

    return segmax


def _epi_body(s_ref, pi_ref, wb3_ref, gamma_ref, beta_ref, o_ref):
    pif = pi_ref[...].astype(jnp.float32)
    cx = (pif[:, 2:3] + 0.5) * 0.2 - 51.2
    cy = (pif[:, 1:2] + 0.5) * 0.2 - 51.2
    cz = jnp.full_like(cx, -1.0)
    centers = jnp.concatenate([cx, cy, cz], axis=1)
    b = lax.dot_general(centers, wb3_ref[...], (((1,), (1,)), ((), ())),
                        precision=lax.Precision.HIGHEST,
                        preferred_element_type=jnp.float32)
    b = b * gamma_ref[...] - beta_ref[...]
    o_ref[...] = jnp.maximum(s_ref[...] - b, 0.0)


def _run_epilogue(s, pillar_indices, wb3, gamma_row, beta_row):
    m = pillar_indices.shape[0]
    bs = 2000
    return pl.pallas_call(
        _epi_body,
        grid=(m // bs,),
        in_specs=[
            pl.BlockSpec((bs, C_OUT), lambda i: (i, 0)),
            pl.BlockSpec((bs, 3), lambda i: (i, 0)),
            pl.BlockSpec((C_OUT, 3), lambda i: (0, 0)),
            pl.BlockSpec((1, C_OUT), lambda i: (0, 0)),
            pl.BlockSpec((1, C_OUT), lambda i: (0, 0)),
        ],
        out_specs=pl.BlockSpec((bs, C_OUT), lambda i: (i, 0)),
        out_shape=jax.ShapeDtypeStruct((m, C_OUT), jnp.float32),
    )(s, pillar_indices, wb3, gamma_row, beta_row)


def kernel(xyz, xyz_batch_cnt, point_features, pillar_indices,
           pillar_set_indices, point_set_indices, W1, gamma1, beta1):
    n = point_features.shape[0]
    m = pillar_indices.shape[0]
    l = pillar_set_indices.shape[0]
    mw = (-(-m // NW) + 7) // 8 * 8
    m_pad = NW * mw
    l_pad = -(-(l + SUP + 8) // SUP) * SUP

    gamma_row = gamma1.reshape(1, C_OUT)
    beta_row = beta1.reshape(1, C_OUT)

    w1tg = W1.T * gamma_row
    wa_blk = jnp.zeros((116, 256), jnp.float32)
    wb_blk = jnp.zeros((12, 256), jnp.float32)
    for k in range(4):
        wa_blk = wa_blk.at[29 * k:29 * k + 29, 64 * k:64 * k + 64].set(w1tg[:29])
        wb_blk = wb_blk.at[3 * k:3 * k + 3, 64 * k:64 * k + 64].set(w1tg[29:])

    pf4 = point_features.reshape(n // 4, 116)
    xyz4 = xyz.reshape(n // 4, 12)
    a = _run_mlp(pf4, xyz4, wa_blk, wb_blk).reshape(n, C_OUT)

    thresholds = jnp.arange(NW + 1, dtype=jnp.int32) * mw
    bounds = jnp.searchsorted(pillar_set_indices, thresholds,
                              side="left").astype(jnp.int32)
    bounds_pad = jnp.zeros((48,), jnp.int32).at[: NW + 1].set(bounds)

    psi_pad = jnp.full((l_pad,), jnp.int32(1 << 29), jnp.int32)
    psi_pad = psi_pad.at[:l].set(pillar_set_indices)
    pidx_pad = jnp.zeros((l_pad,), jnp.int32).at[:l].set(point_set_indices)

    s = _make_segmax(n, l_pad, mw, m_pad)(a, pidx_pad, psi_pad, bounds_pad)

    return _run_epilogue(s, pillar_indices, W1[:, 29:], gamma_row, beta_row)


# speedup vs baseline: 1.2958x; 1.2958x over previous
import functools

import jax
import jax.numpy as jnp
from jax import lax
from jax.experimental import pallas as pl
from jax.experimental.pallas import tpu as pltpu
from jax.experimental.pallas import tpu_sc as plsc

NW = 32
C_OUT = 64
CHUNK = 128
SUP = 2048
SUP_LOG2 = 11
NCH = SUP // CHUNK
NEG = float("-inf")


def _mlp_body(g_ref, w_ref, a_ref):
    g16 = g_ref[...].astype(jnp.bfloat16)
    w16 = w_ref[...].astype(jnp.bfloat16)
    a_ref[...] = lax.dot_general(g16, w16, (((1,), (0,)), ((), ())),
                                 preferred_element_type=jnp.float32)


def _run_mlp(g4, w_blk):
    n4 = g4.shape[0]
    bs = 2000
    return pl.pallas_call(
        _mlp_body,
        grid=(n4 // bs,),
        in_specs=[
            pl.BlockSpec((bs, 128), lambda i: (i, 0)),
            pl.BlockSpec((128, 256), lambda i: (0, 0)),
        ],
        out_specs=pl.BlockSpec((bs, 256), lambda i: (i, 0)),
        out_shape=jax.ShapeDtypeStruct((n4, 256), jnp.float32),
    )(g4, w_blk)


def _make_segmax(n_pts, l_pad, mw, m_pad):
    mesh = plsc.VectorSubcoreMesh(core_axis_name="c", subcore_axis_name="s")

    @functools.partial(
        pl.kernel,
        out_type=jax.ShapeDtypeStruct((m_pad, C_OUT), jnp.float32),
        mesh=mesh,
        compiler_params=pltpu.CompilerParams(use_tc_tiling_on_sc=False),
        scratch_types=[
            pltpu.VMEM((mw + 1, C_OUT), jnp.float32),
            pltpu.VMEM((SUP,), jnp.int32),
            pltpu.VMEM((SUP,), jnp.int32),
            pltpu.VMEM((2, CHUNK, C_OUT), jnp.float32),
            pltpu.VMEM((48,), jnp.int32),
            pltpu.SemaphoreType.DMA,
            pltpu.SemaphoreType.DMA,
            pltpu.SemaphoreType.DMA,
        ],
    )
    def segmax(a_hbm, pidx_hbm, psi_hbm, bounds_hbm, s_hbm,
               acc, pidx_v, psi_v, rows_v, bounds_v, sem_i, sem_g0, sem_g1):
        wid = lax.axis_index("s") * 2 + lax.axis_index("c")
        m0 = wid * mw
        sem_g = (sem_g0, sem_g1)

        neg16 = jnp.full((16,), NEG, jnp.float32)

        def init_row(r, carry):
            for c in range(4):
                acc[r, pl.ds(c * 16, 16)] = neg16
            return carry
        lax.fori_loop(0, mw + 1, init_row, 0, unroll=False)

        pltpu.sync_copy(bounds_hbm, bounds_v)
        bv = bounds_v[pl.ds(wid, 16)]
        lo = bv[0]
        hi = bv[1]
        lo8 = lo & jnp.int32(-8)
        nsup = jnp.maximum((hi - lo8 + (SUP - 1)) >> SUP_LOG2, 1)

        def issue_ids(s):
            base = pl.multiple_of(lo8 + s * SUP, 8)
            pltpu.async_copy(pidx_hbm.at[pl.ds(base, SUP)], pidx_v, sem_i)
            pltpu.async_copy(psi_hbm.at[pl.ds(base, SUP)], psi_v, sem_i)

        def wait_ids():
            pltpu.make_async_copy(pidx_hbm.at[pl.ds(0, SUP)], pidx_v, sem_i).wait()
            pltpu.make_async_copy(psi_hbm.at[pl.ds(0, SUP)], psi_v, sem_i).wait()

        def issue_gather(t, b):
            pltpu.async_copy(a_hbm.at[pidx_v.at[pl.ds(t * CHUNK, CHUNK)]],
                             rows_v.at[b], sem_g[b])

        def wait_gather(b):
            pltpu.make_async_copy(a_hbm.at[pidx_v.at[pl.ds(0, CHUNK)]],
                                  rows_v.at[b], sem_g[b]).wait()

        def compute_chunk(t, b, carry):
            def group(gi, c2):
                rp, m_0, m_1, m_2, m_3 = c2
                sv = psi_v[pl.ds(t * CHUNK + gi * 16, 16)] - m0
                rv = jnp.where((sv < 0) | (sv >= mw), mw, sv)
                for j in range(16):
                    r = rv[j]
                    i = gi * 16 + j
                    row0 = rows_v[b, i, pl.ds(0, 16)]
                    row1 = rows_v[b, i, pl.ds(16, 16)]
                    row2 = rows_v[b, i, pl.ds(32, 16)]
                    row3 = rows_v[b, i, pl.ds(48, 16)]
                    new = r != rp
                    m_0 = jnp.maximum(jnp.where(new, neg16, m_0), row0)
                    m_1 = jnp.maximum(jnp.where(new, neg16, m_1), row1)
                    m_2 = jnp.maximum(jnp.where(new, neg16, m_2), row2)
                    m_3 = jnp.maximum(jnp.where(new, neg16, m_3), row3)
                    acc[r, pl.ds(0, 16)] = m_0
                    acc[r, pl.ds(16, 16)] = m_1
                    acc[r, pl.ds(32, 16)] = m_2
                    acc[r, pl.ds(48, 16)] = m_3
                    rp = r
                return rp, m_0, m_1, m_2, m_3
            return lax.fori_loop(0, CHUNK // 16, group, carry, unroll=False)

        issue_ids(jnp.int32(0))

        def sup_body(s, carry):
            wait_ids()
            issue_gather(0, 0)
            for t in range(NCH):
                if t + 1 < NCH:
                    issue_gather(t + 1, (t + 1) % 2)
                wait_gather(t % 2)
                carry2 = compute_chunk(t, t % 2, carry if t == 0 else carry2)
                carry = carry2

            @pl.when(s + 1 < nsup)
            def _():
                issue_ids(s + 1)
            return carry

        carry0 = (jnp.int32(mw), neg16, neg16, neg16, neg16)
        lax.fori_loop(0, nsup, sup_body, carry0, unroll=False)

        pltpu.sync_copy(acc.at[pl.ds(0, mw)], s_hbm.at[pl.ds(m0, mw)])

    return segmax


def _epi_body(s_ref, pi_ref, wb3_ref, gamma_ref, beta_ref, o_ref):
    pif = pi_ref[...].astype(jnp.float32)
    cx = (pif[:, 2:3] + 0.5) * 0.2 - 51.2
    cy = (pif[:, 1:2] + 0.5) * 0.2 - 51.2
    cz = jnp.full_like(cx, -1.0)
    centers = jnp.concatenate([cx, cy, cz], axis=1)
    b = lax.dot_general(centers, wb3_ref[...], (((1,), (1,)), ((), ())),
                        precision=lax.Precision.HIGHEST,
                        preferred_element_type=jnp.float32)
    b = b * gamma_ref[...] - beta_ref[...]
    o_ref[...] = jnp.maximum(s_ref[...] - b, 0.0)


def _run_epilogue(s, pillar_indices, wb3, gamma_row, beta_row):
    m = pillar_indices.shape[0]
    bs = 2000
    return pl.pallas_call(
        _epi_body,
        grid=(m // bs,),
        in_specs=[
            pl.BlockSpec((bs, C_OUT), lambda i: (i, 0)),
            pl.BlockSpec((bs, 3), lambda i: (i, 0)),
            pl.BlockSpec((C_OUT, 3), lambda i: (0, 0)),
            pl.BlockSpec((1, C_OUT), lambda i: (0, 0)),
            pl.BlockSpec((1, C_OUT), lambda i: (0, 0)),
        ],
        out_specs=pl.BlockSpec((bs, C_OUT), lambda i: (i, 0)),
        out_shape=jax.ShapeDtypeStruct((m, C_OUT), jnp.float32),
    )(s, pillar_indices, wb3, gamma_row, beta_row)


def kernel(xyz, xyz_batch_cnt, point_features, pillar_indices,
           pillar_set_indices, point_set_indices, W1, gamma1, beta1):
    n = point_features.shape[0]
    m = pillar_indices.shape[0]
    l = pillar_set_indices.shape[0]
    mw = (-(-m // NW) + 7) // 8 * 8
    m_pad = NW * mw
    l_pad = -(-(l + SUP + 8) // SUP) * SUP

    gamma_row = gamma1.reshape(1, C_OUT)
    beta_row = beta1.reshape(1, C_OUT)

    w1tg = W1.T * gamma_row
    w_blk = jnp.zeros((128, 256), jnp.float32)
    for k in range(4):
        w_blk = w_blk.at[32 * k:32 * k + 32, 64 * k:64 * k + 64].set(w1tg)

    g4 = jnp.concatenate([point_features, xyz], axis=1).reshape(n // 4, 128)
    a = _run_mlp(g4, w_blk).reshape(n, C_OUT)
    thresholds = jnp.arange(NW + 1, dtype=jnp.int32) * mw
    bounds = jnp.searchsorted(pillar_set_indices, thresholds,
                              side="left").astype(jnp.int32)
    bounds_pad = jnp.zeros((48,), jnp.int32).at[: NW + 1].set(bounds)

    psi_pad = jnp.full((l_pad,), jnp.int32(1 << 29), jnp.int32)
    psi_pad = psi_pad.at[:l].set(pillar_set_indices)
    pidx_pad = jnp.zeros((l_pad,), jnp.int32).at[:l].set(point_set_indices)

    s = _make_segmax(n, l_pad, mw, m_pad)(a, pidx_pad, psi_pad, bounds_pad)

    return _run_epilogue(s, pillar_indices, W1[:, 29:], gamma_row, beta_row)


# speedup vs baseline: 1.3910x; 1.0735x over previous
import functools

import jax
import jax.numpy as jnp
from jax import lax
from jax.experimental import pallas as pl
from jax.experimental.pallas import tpu as pltpu
from jax.experimental.pallas import tpu_sc as plsc

NW = 32
C_OUT = 64
CHUNK = 128
SUP = 2048
SUP_LOG2 = 11
NCH = SUP // CHUNK
NEG = float("-inf")


def _mlp_body(g_ref, w_ref, a_ref):
    g16 = g_ref[...].astype(jnp.bfloat16)
    w16 = w_ref[...].astype(jnp.bfloat16)
    a_ref[...] = lax.dot_general(g16, w16, (((1,), (0,)), ((), ())),
                                 preferred_element_type=jnp.float32
                                 ).astype(jnp.bfloat16)


def _run_mlp(g4, w_blk):
    n4 = g4.shape[0]
    bs = 2000
    return pl.pallas_call(
        _mlp_body,
        grid=(n4 // bs,),
        in_specs=[
            pl.BlockSpec((bs, 128), lambda i: (i, 0)),
            pl.BlockSpec((128, 256), lambda i: (0, 0)),
        ],
        out_specs=pl.BlockSpec((bs, 256), lambda i: (i, 0)),
        out_shape=jax.ShapeDtypeStruct((n4, 256), jnp.bfloat16),
    )(g4, w_blk)


def _make_segmax(n_pts, l_pad, mw, m_pad):
    mesh = plsc.VectorSubcoreMesh(core_axis_name="c", subcore_axis_name="s")

    @functools.partial(
        pl.kernel,
        out_type=jax.ShapeDtypeStruct((m_pad, C_OUT), jnp.bfloat16),
        mesh=mesh,
        compiler_params=pltpu.CompilerParams(use_tc_tiling_on_sc=False),
        scratch_types=[
            pltpu.VMEM((mw + 1, C_OUT), jnp.bfloat16),
            pltpu.VMEM((SUP,), jnp.int32),
            pltpu.VMEM((SUP,), jnp.int32),
            pltpu.VMEM((2, CHUNK, C_OUT), jnp.bfloat16),
            pltpu.VMEM((48,), jnp.int32),
            pltpu.SemaphoreType.DMA,
            pltpu.SemaphoreType.DMA,
            pltpu.SemaphoreType.DMA,
        ],
    )
    def segmax(a_hbm, pidx_hbm, psi_hbm, bounds_hbm, s_hbm,
               acc, pidx_v, psi_v, rows_v, bounds_v, sem_i, sem_g0, sem_g1):
        wid = lax.axis_index("s") * 2 + lax.axis_index("c")
        m0 = wid * mw
        sem_g = (sem_g0, sem_g1)

        neg32 = jnp.full((32,), NEG, jnp.bfloat16)

        def init_row(r, carry):
            for c in range(2):
                acc[r, pl.ds(c * 32, 32)] = neg32
            return carry
        lax.fori_loop(0, mw + 1, init_row, 0, unroll=False)

        pltpu.sync_copy(bounds_hbm, bounds_v)
        bv = bounds_v[pl.ds(wid, 16)]
        lo = bv[0]
        hi = bv[1]
        lo8 = lo & jnp.int32(-8)
        nsup = jnp.maximum((hi - lo8 + (SUP - 1)) >> SUP_LOG2, 1)

        def issue_ids(s):
            base = pl.multiple_of(lo8 + s * SUP, 8)
            pltpu.async_copy(pidx_hbm.at[pl.ds(base, SUP)], pidx_v, sem_i)
            pltpu.async_copy(psi_hbm.at[pl.ds(base, SUP)], psi_v, sem_i)

        def wait_ids():
            pltpu.make_async_copy(pidx_hbm.at[pl.ds(0, SUP)], pidx_v, sem_i).wait()
            pltpu.make_async_copy(psi_hbm.at[pl.ds(0, SUP)], psi_v, sem_i).wait()

        def issue_gather(t, b):
            pltpu.async_copy(a_hbm.at[pidx_v.at[pl.ds(t * CHUNK, CHUNK)]],
                             rows_v.at[b], sem_g[b])

        def wait_gather(b):
            pltpu.make_async_copy(a_hbm.at[pidx_v.at[pl.ds(0, CHUNK)]],
                                  rows_v.at[b], sem_g[b]).wait()

        def compute_chunk(t, b, carry):
            def group(gi, c2):
                rp, m_0, m_1 = c2
                sv = psi_v[pl.ds(t * CHUNK + gi * 16, 16)] - m0
                rv = jnp.where((sv < 0) | (sv >= mw), mw, sv)
                for j in range(16):
                    r = rv[j]
                    i = gi * 16 + j
                    row0 = rows_v[b, i, pl.ds(0, 32)]
                    row1 = rows_v[b, i, pl.ds(32, 32)]
                    new = r != rp
                    m_0 = jnp.maximum(jnp.where(new, neg32, m_0), row0)
                    m_1 = jnp.maximum(jnp.where(new, neg32, m_1), row1)
                    acc[r, pl.ds(0, 32)] = m_0
                    acc[r, pl.ds(32, 32)] = m_1
                    rp = r
                return rp, m_0, m_1
            return lax.fori_loop(0, CHUNK // 16, group, carry, unroll=False)

        issue_ids(jnp.int32(0))

        def sup_body(s, carry):
            wait_ids()
            issue_gather(0, 0)
            for t in range(NCH):
                if t + 1 < NCH:
                    issue_gather(t + 1, (t + 1) % 2)
                wait_gather(t % 2)
                carry2 = compute_chunk(t, t % 2, carry if t == 0 else carry2)
                carry = carry2

            @pl.when(s + 1 < nsup)
            def _():
                issue_ids(s + 1)
            return carry

        carry0 = (jnp.int32(mw), neg32, neg32)
        lax.fori_loop(0, nsup, sup_body, carry0, unroll=False)

        pltpu.sync_copy(acc.at[pl.ds(0, mw)], s_hbm.at[pl.ds(m0, mw)])

    return segmax


def _epi_body(s_ref, pi_ref, wb3_ref, gamma_ref, beta_ref, o_ref):
    pif = pi_ref[...].astype(jnp.float32)
    cx = (pif[:, 2:3] + 0.5) * 0.2 - 51.2
    cy = (pif[:, 1:2] + 0.5) * 0.2 - 51.2
    cz = jnp.full_like(cx, -1.0)
    centers = jnp.concatenate([cx, cy, cz], axis=1)
    b = lax.dot_general(centers, wb3_ref[...], (((1,), (1,)), ((), ())),
                        precision=lax.Precision.HIGHEST,
                        preferred_element_type=jnp.float32)
    b = b * gamma_ref[...] - beta_ref[...]
    o_ref[...] = jnp.maximum(s_ref[...].astype(jnp.float32) - b, 0.0)


def _run_epilogue(s, pillar_indices, wb3, gamma_row, beta_row):
    m = pillar_indices.shape[0]
    bs = 2000
    return pl.pallas_call(
        _epi_body,
        grid=(m // bs,),
        in_specs=[
            pl.BlockSpec((bs, C_OUT), lambda i: (i, 0)),
            pl.BlockSpec((bs, 3), lambda i: (i, 0)),
            pl.BlockSpec((C_OUT, 3), lambda i: (0, 0)),
            pl.BlockSpec((1, C_OUT), lambda i: (0, 0)),
            pl.BlockSpec((1, C_OUT), lambda i: (0, 0)),
        ],
        out_specs=pl.BlockSpec((bs, C_OUT), lambda i: (i, 0)),
        out_shape=jax.ShapeDtypeStruct((m, C_OUT), jnp.float32),
    )(s, pillar_indices, wb3, gamma_row, beta_row)


def kernel(xyz, xyz_batch_cnt, point_features, pillar_indices,
           pillar_set_indices, point_set_indices, W1, gamma1, beta1):
    n = point_features.shape[0]
    m = pillar_indices.shape[0]
    l = pillar_set_indices.shape[0]
    mw = (-(-m // NW) + 7) // 8 * 8
    m_pad = NW * mw
    l_pad = -(-(l + SUP + 8) // SUP) * SUP

    gamma_row = gamma1.reshape(1, C_OUT)
    beta_row = beta1.reshape(1, C_OUT)

    w1tg = W1.T * gamma_row
    w_blk = jnp.zeros((128, 256), jnp.float32)
    for k in range(4):
        w_blk = w_blk.at[32 * k:32 * k + 32, 64 * k:64 * k + 64].set(w1tg)

    g4 = jnp.concatenate([point_features, xyz], axis=1).reshape(n // 4, 128)
    a = _run_mlp(g4, w_blk).reshape(n, C_OUT)
    thresholds = jnp.arange(NW + 1, dtype=jnp.int32) * mw
    bounds = jnp.searchsorted(pillar_set_indices, thresholds,
                              side="left").astype(jnp.int32)
    bounds_pad = jnp.zeros((48,), jnp.int32).at[: NW + 1].set(bounds)

    psi_pad = jnp.full((l_pad,), jnp.int32(1 << 29), jnp.int32)
    psi_pad = psi_pad.at[:l].set(pillar_set_indices)
    pidx_pad = jnp.zeros((l_pad,), jnp.int32).at[:l].set(point_set_indices)

    s = _make_segmax(n, l_pad, mw, m_pad)(a, pidx_pad, psi_pad, bounds_pad)

    return _run_epilogue(s, pillar_indices, W1[:, 29:], gamma_row, beta_row)


# speedup vs baseline: 1.4141x; 1.0167x over previous
import functools

import jax
import jax.numpy as jnp
from jax import lax
from jax.experimental import pallas as pl
from jax.experimental.pallas import tpu as pltpu
from jax.experimental.pallas import tpu_sc as plsc

NW = 32
C_OUT = 64
CHUNK = 128
SUP = 2048
SUP_LOG2 = 11
NCH = SUP // CHUNK
NEG = float("-inf")


def _mlp_body(g_ref, w_ref, a_ref):
    g16 = g_ref[...].astype(jnp.bfloat16)
    w16 = w_ref[...].astype(jnp.bfloat16)
    a_ref[...] = lax.dot_general(g16, w16, (((1,), (0,)), ((), ())),
                                 preferred_element_type=jnp.float32
                                 ).astype(jnp.bfloat16)


def _run_mlp(g4, w_blk):
    n4 = g4.shape[0]
    bs = 2000
    return pl.pallas_call(
        _mlp_body,
        grid=(n4 // bs,),
        in_specs=[
            pl.BlockSpec((bs, 128), lambda i: (i, 0)),
            pl.BlockSpec((128, 256), lambda i: (0, 0)),
        ],
        out_specs=pl.BlockSpec((bs, 256), lambda i: (i, 0)),
        out_shape=jax.ShapeDtypeStruct((n4, 256), jnp.bfloat16),
    )(g4, w_blk)


def _make_segmax(n_pts, l_pad, mw, m_pad):
    mesh = plsc.VectorSubcoreMesh(core_axis_name="c", subcore_axis_name="s")

    @functools.partial(
        pl.kernel,
        out_type=jax.ShapeDtypeStruct((m_pad, C_OUT), jnp.bfloat16),
        mesh=mesh,
        compiler_params=pltpu.CompilerParams(use_tc_tiling_on_sc=False),
        scratch_types=[
            pltpu.VMEM((mw + 1, C_OUT), jnp.bfloat16),
            pltpu.VMEM((SUP,), jnp.int32),
            pltpu.VMEM((SUP,), jnp.int32),
            pltpu.VMEM((2, 2 * CHUNK, C_OUT), jnp.bfloat16),
            pltpu.VMEM((48,), jnp.int32),
            pltpu.SemaphoreType.DMA,
            pltpu.SemaphoreType.DMA,
            pltpu.SemaphoreType.DMA,
        ],
    )
    def segmax(a_hbm, pidx_hbm, psi_hbm, bounds_hbm, s_hbm,
               acc, pidx_v, psi_v, rows_v, bounds_v, sem_i, sem_g0, sem_g1):
        wid = lax.axis_index("s") * 2 + lax.axis_index("c")
        m0 = wid * mw
        sem_g = (sem_g0, sem_g1)

        neg32 = jnp.full((32,), NEG, jnp.bfloat16)

        def init_row(r, carry):
            for c in range(2):
                acc[r, pl.ds(c * 32, 32)] = neg32
            return carry
        lax.fori_loop(0, mw + 1, init_row, 0, unroll=False)

        pltpu.sync_copy(bounds_hbm, bounds_v)
        bv = bounds_v[pl.ds(wid, 16)]
        lo = bv[0]
        hi = bv[1]
        lo8 = lo & jnp.int32(-8)
        nsup = jnp.maximum((hi - lo8 + (SUP - 1)) >> SUP_LOG2, 1)

        def issue_ids(s):
            base = pl.multiple_of(lo8 + s * SUP, 8)
            pltpu.async_copy(pidx_hbm.at[pl.ds(base, SUP)], pidx_v, sem_i)
            pltpu.async_copy(psi_hbm.at[pl.ds(base, SUP)], psi_v, sem_i)

        def wait_ids():
            pltpu.make_async_copy(pidx_hbm.at[pl.ds(0, SUP)], pidx_v, sem_i).wait()
            pltpu.make_async_copy(psi_hbm.at[pl.ds(0, SUP)], psi_v, sem_i).wait()

        def issue_gather(t, b):
            pltpu.async_copy(
                a_hbm.at[pidx_v.at[pl.ds(2 * t * CHUNK, CHUNK)]],
                rows_v.at[b, pl.ds(0, CHUNK)], sem_g[b])
            pltpu.async_copy(
                a_hbm.at[pidx_v.at[pl.ds((2 * t + 1) * CHUNK, CHUNK)]],
                rows_v.at[b, pl.ds(CHUNK, CHUNK)], sem_g[b])

        def wait_gather(b):
            pltpu.make_async_copy(a_hbm.at[pidx_v.at[pl.ds(0, 2 * CHUNK)]],
                                  rows_v.at[b], sem_g[b]).wait()

        def compute_chunk(t, b, carry):
            def group(gi, c2):
                rp, m_0, m_1 = c2
                sv = psi_v[pl.ds(2 * t * CHUNK + gi * 16, 16)] - m0
                rv = jnp.where((sv < 0) | (sv >= mw), mw, sv)
                for j in range(16):
                    r = rv[j]
                    i = gi * 16 + j
                    row0 = rows_v[b, i, pl.ds(0, 32)]
                    row1 = rows_v[b, i, pl.ds(32, 32)]
                    new = r != rp
                    m_0 = jnp.maximum(jnp.where(new, neg32, m_0), row0)
                    m_1 = jnp.maximum(jnp.where(new, neg32, m_1), row1)
                    acc[r, pl.ds(0, 32)] = m_0
                    acc[r, pl.ds(32, 32)] = m_1
                    rp = r
                return rp, m_0, m_1
            return lax.fori_loop(0, 2 * CHUNK // 16, group, carry, unroll=False)

        issue_ids(jnp.int32(0))

        def sup_body(s, carry):
            wait_ids()
            issue_gather(0, 0)
            for t in range(NCH // 2):
                if t + 1 < NCH // 2:
                    issue_gather(t + 1, (t + 1) % 2)
                wait_gather(t % 2)
                carry2 = compute_chunk(t, t % 2, carry if t == 0 else carry2)
                carry = carry2

            @pl.when(s + 1 < nsup)
            def _():
                issue_ids(s + 1)
            return carry

        carry0 = (jnp.int32(mw), neg32, neg32)
        lax.fori_loop(0, nsup, sup_body, carry0, unroll=False)

        pltpu.sync_copy(acc.at[pl.ds(0, mw)], s_hbm.at[pl.ds(m0, mw)])

    return segmax


def _epi_body(s_ref, pi_ref, wb3_ref, gamma_ref, beta_ref, o_ref):
    pif = pi_ref[...].astype(jnp.float32)
    cx = (pif[:, 2:3] + 0.5) * 0.2 - 51.2
    cy = (pif[:, 1:2] + 0.5) * 0.2 - 51.2
    cz = jnp.full_like(cx, -1.0)
    centers = jnp.concatenate([cx, cy, cz], axis=1)
    b = lax.dot_general(centers, wb3_ref[...], (((1,), (1,)), ((), ())),
                        precision=lax.Precision.HIGHEST,
                        preferred_element_type=jnp.float32)
    b = b * gamma_ref[...] - beta_ref[...]
    o_ref[...] = jnp.maximum(s_ref[...].astype(jnp.float32) - b, 0.0)


def _run_epilogue(s, pillar_indices, wb3, gamma_row, beta_row):
    m = pillar_indices.shape[0]
    bs = 2000
    return pl.pallas_call(
        _epi_body,
        grid=(m // bs,),
        in_specs=[
            pl.BlockSpec((bs, C_OUT), lambda i: (i, 0)),
            pl.BlockSpec((bs, 3), lambda i: (i, 0)),
            pl.BlockSpec((C_OUT, 3), lambda i: (0, 0)),
            pl.BlockSpec((1, C_OUT), lambda i: (0, 0)),
            pl.BlockSpec((1, C_OUT), lambda i: (0, 0)),
        ],
        out_specs=pl.BlockSpec((bs, C_OUT), lambda i: (i, 0)),
        out_shape=jax.ShapeDtypeStruct((m, C_OUT), jnp.float32),
    )(s, pillar_indices, wb3, gamma_row, beta_row)


def kernel(xyz, xyz_batch_cnt, point_features, pillar_indices,
           pillar_set_indices, point_set_indices, W1, gamma1, beta1):
    n = point_features.shape[0]
    m = pillar_indices.shape[0]
    l = pillar_set_indices.shape[0]
    mw = (-(-m // NW) + 7) // 8 * 8
    m_pad = NW * mw
    l_pad = -(-(l + SUP + 8) // SUP) * SUP

    gamma_row = gamma1.reshape(1, C_OUT)
    beta_row = beta1.reshape(1, C_OUT)

    w1tg = W1.T * gamma_row
    w_blk = jnp.zeros((128, 256), jnp.float32)
    for k in range(4):
        w_blk = w_blk.at[32 * k:32 * k + 32, 64 * k:64 * k + 64].set(w1tg)

    g4 = jnp.concatenate([point_features, xyz], axis=1).reshape(n // 4, 128)
    a = _run_mlp(g4, w_blk).reshape(n, C_OUT)
    thresholds = jnp.arange(NW + 1, dtype=jnp.int32) * mw
    bounds = jnp.searchsorted(pillar_set_indices, thresholds,
                              side="left").astype(jnp.int32)
    bounds_pad = jnp.zeros((48,), jnp.int32).at[: NW + 1].set(bounds)

    psi_pad = jnp.full((l_pad,), jnp.int32(1 << 29), jnp.int32)
    psi_pad = psi_pad.at[:l].set(pillar_set_indices)
    pidx_pad = jnp.zeros((l_pad,), jnp.int32).at[:l].set(point_set_indices)

    s = _make_segmax(n, l_pad, mw, m_pad)(a, pidx_pad, psi_pad, bounds_pad)

    return _run_epilogue(s, pillar_indices, W1[:, 29:], gamma_row, beta_row)
